# baseline (speedup 1.0000x reference)
import functools

import jax
import jax.numpy as jnp
import numpy as np
from jax.experimental import pallas as pl
from jax.experimental.pallas import tpu as pltpu
from jax.experimental.pallas import tpu_sc as plsc

B, S, D = 1, 2048, 768
H, DH = 12, 64
DI = 1536
FF = 2048
E = 8
CAP = 1024
EPS = 1e-6
SBLK = 128
NSB = S // SBLK
GCOL = S // SBLK
PBLK = 128
NPB = 24
PSLOTS = NPB * PBLK
TRASH = 128
JT = 256
HPS = 6
F32 = jnp.float32


def _rmsnorm(x, w):
    return x * jax.lax.rsqrt(jnp.mean(x * x, axis=-1, keepdims=True) + EPS) * w


def _cumsum_iorder(x):
    r = jax.lax.broadcasted_iota(jnp.int32, (SBLK, SBLK), 0)
    c = jax.lax.broadcasted_iota(jnp.int32, (SBLK, SBLK), 1)
    tril = (r >= c).astype(F32)
    col = jnp.dot(tril, x, preferred_element_type=F32)
    g = x.shape[1]
    rg = jax.lax.broadcasted_iota(jnp.int32, (g, g), 0)
    cg = jax.lax.broadcasted_iota(jnp.int32, (g, g), 1)
    up = (rg < cg).astype(F32)
    tot = jnp.sum(x, axis=0, keepdims=True)
    pref = jnp.dot(tot, up, preferred_element_type=F32)
    return col + pref


def _ssm_body(h_ref, rms_ref, win_ref, wdt_ref, bdt_ref, alog_ref, wout_ref,
              wmod_ref, h1_ref, wsel_ref, src_ref, aux_ref,
              carry, r_s, winb, wdtb, woutb):
    i = pl.program_id(0)

    @pl.when(i == 0)
    def _():
        carry[...] = jnp.zeros_like(carry)
        winb[...] = win_ref[...].astype(jnp.bfloat16)
        wdtb[...] = wdt_ref[...].astype(jnp.bfloat16)
        woutb[...] = wout_ref[...].astype(jnp.bfloat16)

    @pl.when(i < NSB)
    def _():
        h = h_ref[...]
        xn = _rmsnorm(h, rms_ref[...])
        xnb = xn.astype(jnp.bfloat16)
        xz = jnp.dot(xnb, winb[...], preferred_element_type=F32)
        x_in = xz[:, :DI]
        z = xz[:, DI:]
        delta = jax.nn.softplus(
            jnp.dot(xnb, wdtb[...], preferred_element_type=F32)
            + bdt_ref[...])
        decay = jnp.exp(delta * (-jnp.exp(alog_ref[...])))
        u = delta * x_in

        a, b = decay, u
        d = 1
        while d < SBLK:
            a_s = jnp.concatenate([jnp.ones((d, DI), F32), a[:-d]], axis=0)
            b_s = jnp.concatenate([jnp.zeros((d, DI), F32), b[:-d]], axis=0)
            b = b + a * b_s
            a = a * a_s
            d *= 2
        s = b + a * carry[...]
        carry[...] = s[SBLK - 1:, :]

        h1 = h + jnp.dot((s * jax.nn.silu(z)).astype(jnp.bfloat16),
                         woutb[...], preferred_element_type=F32)
        h1_ref[...] = h1
        rcol = jnp.dot(h1, wmod_ref[...], preferred_element_type=F32)
        for j in range(NSB):
            @pl.when(i == j)
            def _(j=j):
                r_s[:, j:j + 1] = rcol

    @pl.when(i == NSB)
    def _():
        _router_tail(r_s[...], wsel_ref, src_ref, aux_ref)


def _run_ssm(h, rms_ssm, w_in, w_dt, b_dt, a_log, w_out, w_mod):
    full = lambda shp: pl.BlockSpec(shp, lambda i: (0, 0))
    blk = lambda i: (jnp.minimum(i, NSB - 1), 0)
    return pl.pallas_call(
        _ssm_body,
        grid=(NSB + 1,),
        in_specs=[
            pl.BlockSpec((SBLK, D), blk),
            full((1, D)), full((D, 2 * DI)), full((D, DI)), full((1, DI)),
            full((1, DI)), full((DI, D)), full((D, 1)),
        ],
        out_specs=[
            pl.BlockSpec((SBLK, D), blk),
            full((1, CAP)), full((SBLK, GCOL)), full((1, 1)),
        ],
        out_shape=[
            jax.ShapeDtypeStruct((S, D), F32),
            jax.ShapeDtypeStruct((1, CAP), F32),
            jax.ShapeDtypeStruct((SBLK, GCOL), jnp.int32),
            jax.ShapeDtypeStruct((1, 1), F32),
        ],
        scratch_shapes=[pltpu.VMEM((1, DI), F32),
                        pltpu.VMEM((SBLK, NSB), F32),
                        pltpu.VMEM((D, 2 * DI), jnp.bfloat16),
                        pltpu.VMEM((D, DI), jnp.bfloat16),
                        pltpu.VMEM((DI, D), jnp.bfloat16)],
    )(h, rms_ssm.reshape(1, D), w_in, w_dt, b_dt.reshape(1, DI),
      a_log.reshape(1, DI), w_out, w_mod.reshape(D, 1))


def _router_tail(r2, wsel_ref, src_ref, aux_ref):
    u = jax.lax.bitcast_convert_type(r2, jnp.int32)
    keys = u ^ (jax.lax.shift_right_arithmetic(u, 31) & jnp.int32(0x7FFFFFFF))

    def bs_body(_, lh):
        lo, hi = lh
        mid = (lo >> 1) + (hi >> 1) + (lo & hi & 1)
        umid = mid + ((lo ^ hi) & 1)
        cnt = jnp.sum((keys >= umid).astype(jnp.int32))
        ok = cnt >= CAP
        return (jnp.where(ok, umid, lo), jnp.where(ok, hi, umid - 1))

    lo, _ = jax.lax.fori_loop(
        0, 32, bs_body,
        (jnp.int32(np.int32(-2**31)), jnp.int32(np.int32(2**31 - 1))))
    thr = lo

    gt = (keys > thr).astype(F32)
    eq = (keys == thr).astype(F32)
    need = jnp.float32(CAP) - jnp.sum(gt)
    rank_eq = _cumsum_iorder(eq)
    sel = gt + eq * (rank_eq <= need).astype(F32)
    pos = _cumsum_iorder(sel) - 1.0

    icol = (jax.lax.broadcasted_iota(jnp.int32, (SBLK, GCOL), 0)
            + SBLK * jax.lax.broadcasted_iota(jnp.int32, (SBLK, GCOL), 1))
    src_ref[...] = jnp.where(sel > 0.5,
                             jnp.int32(S) + pos.astype(jnp.int32), icol)

    sigv = jax.nn.sigmoid(r2)
    for jt in range(CAP // JT):
        jio = (jax.lax.broadcasted_iota(jnp.int32, (1, JT), 1).astype(F32)
               + jnp.float32(jt * JT))
        acc_w = jnp.zeros((1, JT), F32)
        for g in range(GCOL):
            m = (pos[:, g:g + 1] == jio).astype(F32) * sel[:, g:g + 1]
            acc_w = acc_w + jnp.sum(m * sigv[:, g:g + 1], axis=0,
                                    keepdims=True)
        wsel_ref[0:1, jt * JT:(jt + 1) * JT] = acc_w

    aux_ref[...] = jnp.sum(sigv).reshape(1, 1) / jnp.float32(S)


_NW = 32


def _sc_gather_rows(data, idx_row):
    m = idx_row.shape[1]
    bpw = m // _NW
    mesh = plsc.VectorSubcoreMesh(core_axis_name="c", subcore_axis_name="s")

    @functools.partial(
        pl.kernel,
        out_type=jax.ShapeDtypeStruct((m, D), F32),
        mesh=mesh,
        scratch_types=[
            pltpu.VMEM((bpw,), jnp.int32),
            pltpu.VMEM((bpw, D), F32),
            pltpu.SemaphoreType.DMA,
        ])
    def k(x_hbm, i_hbm, o_hbm, idx_v, rows_v, sem):
        wid = jax.lax.axis_index("s") * 2 + jax.lax.axis_index("c")
        base = wid * bpw
        pltpu.sync_copy(i_hbm.at[pl.ds(base, bpw)], idx_v)
        pltpu.async_copy(x_hbm.at[idx_v], rows_v, sem).wait()
        pltpu.sync_copy(rows_v, o_hbm.at[pl.ds(base, bpw)])

    return k(data, idx_row.reshape(m))


def _sc_gather_rows2(data1, data2, idx_row):
    m = idx_row.shape[1]
    bpw = m // _NW
    mesh = plsc.VectorSubcoreMesh(core_axis_name="c", subcore_axis_name="s")

    @functools.partial(
        pl.kernel,
        out_type=(jax.ShapeDtypeStruct((m, D), F32),
                  jax.ShapeDtypeStruct((m, D), F32)),
        mesh=mesh,
        scratch_types=[
            pltpu.VMEM((bpw,), jnp.int32),
            pltpu.VMEM((bpw, D), F32),
            pltpu.SemaphoreType.DMA,
        ])
    def k(x1_hbm, x2_hbm, i_hbm, o1_hbm, o2_hbm, idx_v, rows_v, sem):
        wid = jax.lax.axis_index("s") * 2 + jax.lax.axis_index("c")
        base = wid * bpw
        pltpu.sync_copy(i_hbm.at[pl.ds(base, bpw)], idx_v)
        pltpu.async_copy(x1_hbm.at[idx_v], rows_v, sem).wait()
        pltpu.sync_copy(rows_v, o1_hbm.at[pl.ds(base, bpw)])
        pltpu.async_copy(x2_hbm.at[idx_v], rows_v, sem).wait()
        pltpu.sync_copy(rows_v, o2_hbm.at[pl.ds(base, bpw)])

    return k(data1, data2, idx_row.reshape(m))


def _sc_scatter_rows(values, idx_row, out_rows):
    m = idx_row.shape[1]
    bpw = m // _NW
    mesh = plsc.VectorSubcoreMesh(core_axis_name="c", subcore_axis_name="s")

    @functools.partial(
        pl.kernel,
        out_type=jax.ShapeDtypeStruct((out_rows, D), F32),
        mesh=mesh,
        scratch_types=[
            pltpu.VMEM((bpw,), jnp.int32),
            pltpu.VMEM((bpw, D), F32),
            pltpu.SemaphoreType.DMA,
        ])
    def k(x_hbm, i_hbm, o_hbm, idx_v, rows_v, sem):
        wid = jax.lax.axis_index("s") * 2 + jax.lax.axis_index("c")
        base = wid * bpw
        pltpu.sync_copy(i_hbm.at[pl.ds(base, bpw)], idx_v)
        pltpu.sync_copy(x_hbm.at[pl.ds(base, bpw)], rows_v)
        pltpu.async_copy(rows_v, o_hbm.at[idx_v], sem).wait()

    return k(values, idx_row.reshape(m))


def _attn_body(h1_ref, src_ref, rms_ref, wq_ref, wk_ref, wv_ref, wo_ref,
               wsel_ref, delta_ref, an_s, acc_s):
    h = pl.program_id(0)

    @pl.when(h == 0)
    def _():
        jrow = jax.lax.broadcasted_iota(jnp.int32, (1, CAP), 1) + jnp.int32(S)
        cols = [
            (src_ref[t] == jrow).astype(jnp.bfloat16) for t in range(GCOL)
        ]
        n = jnp.concatenate(cols, axis=0)
        selb = jax.lax.dot_general(
            n, h1_ref[...].astype(jnp.bfloat16), (((0,), (0,)), ((), ())),
            preferred_element_type=F32)
        an_s[...] = _rmsnorm(selb, rms_ref[...])
        acc_s[...] = jnp.zeros_like(acc_s)

    an = an_s[...].astype(jnp.bfloat16)
    scale = jnp.float32(1.0 / np.sqrt(DH))
    negtri = jnp.where(
        jax.lax.broadcasted_iota(jnp.int32, (SBLK, SBLK), 0)
        >= jax.lax.broadcasted_iota(jnp.int32, (SBLK, SBLK), 1),
        jnp.float32(0.0), jnp.float32(-1e9))
    q2 = jnp.dot(an, wq_ref[...].astype(jnp.bfloat16),
                 preferred_element_type=F32).astype(jnp.bfloat16)
    k2 = jnp.dot(an, wk_ref[...].astype(jnp.bfloat16),
                 preferred_element_type=F32).astype(jnp.bfloat16)
    v2 = jnp.dot(an, wv_ref[...].astype(jnp.bfloat16),
                 preferred_element_type=F32).astype(jnp.bfloat16)
    for j in range(HPS):
        wo_b = wo_ref[j * DH:(j + 1) * DH, :].astype(jnp.bfloat16)
        q = q2[:, j * DH:(j + 1) * DH]
        k = k2[:, j * DH:(j + 1) * DH]
        v = v2[:, j * DH:(j + 1) * DH]
        for qb in range(CAP // SBLK):
            kw = (qb + 1) * SBLK
            qq = q[qb * SBLK:(qb + 1) * SBLK]
            s = jax.lax.dot_general(qq, k[:kw], (((1,), (1,)), ((), ())),
                                    preferred_element_type=F32) * scale
            if qb == 0:
                s = s + negtri
            else:
                s = jnp.concatenate(
                    [s[:, :kw - SBLK], s[:, kw - SBLK:] + negtri], axis=1)
            mx = jnp.max(s, axis=-1, keepdims=True)
            p = jnp.exp(s - mx)
            inv = jnp.float32(1.0) / jnp.sum(p, axis=-1, keepdims=True)
            o = jnp.dot(p.astype(jnp.bfloat16), v[:kw],
                        preferred_element_type=F32) * inv
            acc_s[qb * SBLK:(qb + 1) * SBLK, :] += jnp.dot(
                o.astype(jnp.bfloat16), wo_b, preferred_element_type=F32)

    @pl.when(h == H // HPS - 1)
    def _():
        delta_ref[...] = acc_s[...] * wsel_ref[...]


def _run_attn(h1, src3, rms_attn, wq, wk, wv, wo, wsel_col):
    full = lambda shp: pl.BlockSpec(shp, lambda h: (0, 0))
    return pl.pallas_call(
        _attn_body,
        grid=(H // HPS,),
        in_specs=[
            full((S, D)),
            pl.BlockSpec((GCOL, SBLK, 1), lambda h: (0, 0, 0)),
            full((1, D)),
            pl.BlockSpec((D, HPS * DH), lambda h: (0, h)),
            pl.BlockSpec((D, HPS * DH), lambda h: (0, h)),
            pl.BlockSpec((D, HPS * DH), lambda h: (0, h)),
            pl.BlockSpec((HPS * DH, D), lambda h: (h, 0)),
            full((CAP, 1)),
        ],
        out_specs=full((CAP, D)),
        out_shape=jax.ShapeDtypeStruct((CAP, D), F32),
        scratch_shapes=[pltpu.VMEM((CAP, D), F32), pltpu.VMEM((CAP, D), F32)],
    )(h1, src3, rms_attn.reshape(1, D), wq, wk, wv, wo, wsel_col)


def _gate_body(h1_ref, delta_ref, src_ref, rms_ref, wg_ref, mn_ref, h2_ref,
               tokg_ref, toks_ref, gates_ref, be_ref, aux_ref,
               pe_s, fe_s, e_s, g_s):
    i = pl.program_id(0)

    @pl.when(i == 0)
    def _():
        pe_s[...] = jnp.zeros_like(pe_s)
        fe_s[...] = jnp.zeros_like(fe_s)

    @pl.when(i < NSB)
    def _():
        jrow = jax.lax.broadcasted_iota(jnp.int32, (1, CAP), 1) + jnp.int32(S)
        mcol = (src_ref[0] == jrow).astype(jnp.bfloat16)
        h2 = h1_ref[...] + jnp.dot(mcol, delta_ref[...].astype(jnp.bfloat16),
                                   preferred_element_type=F32)
        h2_ref[...] = h2
        mn = _rmsnorm(h2, rms_ref[...])
        mn_ref[...] = mn
        logits = jnp.dot(mn, wg_ref[...], preferred_element_type=F32)
        mx = jnp.max(logits, axis=-1, keepdims=True)
        ex = jnp.exp(logits - mx)
        probs = ex / jnp.sum(ex, axis=-1, keepdims=True)
        g = jnp.max(probs, axis=-1, keepdims=True)
        ei = jax.lax.broadcasted_iota(jnp.int32, (SBLK, E), 1)
        eid = jnp.min(jnp.where(probs >= g, ei, jnp.int32(E)), axis=-1,
                      keepdims=True)
        for j in range(NSB):
            @pl.when(i == j)
            def _(j=j):
                e_s[:, j:j + 1] = eid
                g_s[:, j:j + 1] = g
        pe_s[...] += jnp.sum(probs, axis=0, keepdims=True)
        fe_s[...] += jnp.sum((ei == eid).astype(F32), axis=0, keepdims=True)

    @pl.when(i == NSB)
    def _():
        aux_ref[...] = (jnp.float32(E) / jnp.float32(S * S)
                        * jnp.sum(fe_s[...] * pe_s[...])).reshape(1, 1)
        _route_tail(e_s[...], g_s[...], tokg_ref, toks_ref, gates_ref, be_ref)


def _run_gate(h1, delta, src3, rms_moe, w_gate):
    full = lambda shp: pl.BlockSpec(shp, lambda i: (0, 0))
    blk = lambda i: (jnp.minimum(i, NSB - 1), 0)
    return pl.pallas_call(
        _gate_body,
        grid=(NSB + 1,),
        in_specs=[pl.BlockSpec((SBLK, D), blk),
                  full((CAP, D)),
                  pl.BlockSpec((1, SBLK, 1), lambda i: (jnp.minimum(i, NSB - 1), 0, 0)),
                  full((1, D)), full((D, E))],
        out_specs=[
            pl.BlockSpec((SBLK, D), blk),
            pl.BlockSpec((SBLK, D), blk),
            full((1, PSLOTS)), full((1, PSLOTS)), full((1, PSLOTS)),
            full((1, NPB)), full((1, 1)),
        ],
        out_shape=[
            jax.ShapeDtypeStruct((S, D), F32),
            jax.ShapeDtypeStruct((S, D), F32),
            jax.ShapeDtypeStruct((1, PSLOTS), jnp.int32),
            jax.ShapeDtypeStruct((1, PSLOTS), jnp.int32),
            jax.ShapeDtypeStruct((1, PSLOTS), F32),
            jax.ShapeDtypeStruct((1, NPB), jnp.int32),
            jax.ShapeDtypeStruct((1, 1), F32),
        ],
        scratch_shapes=[pltpu.VMEM((1, E), F32), pltpu.VMEM((1, E), F32),
                        pltpu.VMEM((SBLK, NSB), jnp.int32),
                        pltpu.VMEM((SBLK, NSB), F32)],
    )(h1, delta, src3, rms_moe.reshape(1, D), w_gate)


def _route_tail(ecol, gcol, tokg_ref, toks_ref, gates_ref, be_ref):
    slot = jnp.zeros((SBLK, GCOL), F32)
    starts, ends, real_ends = [], [], []
    off = jnp.int32(0)
    for e in range(E):
        m = (ecol == e).astype(F32)
        rank = _cumsum_iorder(m)
        cnt = jnp.sum(m).astype(jnp.int32)
        slot = slot + m * (jnp.float32(1.0) * off + rank - 1.0)
        starts.append(off)
        real_ends.append(off + cnt)
        off = off + ((cnt + PBLK - 1) // PBLK) * PBLK
        ends.append(off)

    bio = jax.lax.broadcasted_iota(jnp.int32, (1, NPB), 1)
    be = jnp.full((1, NPB), E - 1, jnp.int32)
    for e in range(E - 1):
        inb = (bio >= starts[e] // PBLK) & (bio < ends[e] // PBLK)
        be = be - jnp.int32(E - 1 - e) * inb.astype(jnp.int32)
    be_ref[...] = be

    for jt in range(PSLOTS // JT):
        jioi = (jax.lax.broadcasted_iota(jnp.int32, (1, JT), 1)
                + jnp.int32(jt * JT))
        jio = jioi.astype(F32)
        acc_t = jnp.zeros((1, JT), F32)
        acc_g = jnp.zeros((1, JT), F32)
        for g in range(GCOL):
            m = (slot[:, g:g + 1] == jio).astype(F32)
            gidx = (jax.lax.broadcasted_iota(jnp.int32, (SBLK, 1), 0)
                    .astype(F32) + jnp.float32(g * SBLK))
            acc_t = acc_t + jnp.sum(m * gidx, axis=0, keepdims=True)
            acc_g = acc_g + jnp.sum(m * gcol[:, g:g + 1], axis=0,
                                    keepdims=True)
        covered = jnp.zeros((1, JT), jnp.bool_)
        for e in range(E):
            covered = covered | ((jioi >= starts[e]) & (jioi < real_ends[e]))
        tok = acc_t.astype(jnp.int32)
        sl = slice(jt * JT, (jt + 1) * JT)
        tokg_ref[0:1, sl] = jnp.where(covered, tok, jioi % S)
        toks_ref[0:1, sl] = jnp.where(covered, tok,
                                      jnp.int32(S) + (jioi % TRASH))
        gates_ref[0:1, sl] = acc_g


def _expert_body(be_ref, x_ref, res_ref, g_ref, wup_ref, wdn_ref, o_ref):
    x = x_ref[...].astype(jnp.bfloat16)
    hmid = jax.nn.silu(jnp.dot(x, wup_ref[0].astype(jnp.bfloat16),
                               preferred_element_type=F32))
    o = jnp.dot(hmid.astype(jnp.bfloat16), wdn_ref[0].astype(jnp.bfloat16),
                preferred_element_type=F32)
    o_ref[...] = res_ref[...] + o * g_ref[...]


def _run_experts(x_moe, x_res, gates_col, w_up, w_down, block_expert):
    spec = pltpu.PrefetchScalarGridSpec(
        num_scalar_prefetch=1,
        grid=(NPB,),
        in_specs=[
            pl.BlockSpec((PBLK, D), lambda i, be: (i, 0)),
            pl.BlockSpec((PBLK, D), lambda i, be: (i, 0)),
            pl.BlockSpec((PBLK, 1), lambda i, be: (i, 0)),
            pl.BlockSpec((1, D, FF), lambda i, be: (be[i], 0, 0)),
            pl.BlockSpec((1, FF, D), lambda i, be: (be[i], 0, 0)),
        ],
        out_specs=pl.BlockSpec((PBLK, D), lambda i, be: (i, 0)),
    )
    return pl.pallas_call(
        _expert_body,
        grid_spec=spec,
        out_shape=jax.ShapeDtypeStruct((PSLOTS, D), F32),
    )(block_expert, x_moe, x_res, gates_col, w_up, w_down)


def kernel(hidden_states, rms_ssm, W_in, W_dt, b_dt, A_log, W_out_ssm, w_mod,
           rms_attn, Wq, Wk, Wv, Wo, rms_moe, W_gate, W_up, W_down):
    h = hidden_states.reshape(S, D)

    h1, wsel_row, src_col, aux1 = _run_ssm(
        h, rms_ssm, W_in, W_dt, b_dt, A_log, W_out_ssm, w_mod)

    src3 = src_col.T.reshape(GCOL, SBLK, 1)
    delta = _run_attn(h1, src3, rms_attn, Wq, Wk, Wv, Wo,
                      wsel_row.reshape(CAP, 1))
    mn, h2, tokg, toks, gates, block_expert, aux2 = _run_gate(
        h1, delta, src3, rms_moe, W_gate)

    x_moe, x_res = _sc_gather_rows2(mn, h2, tokg)
    y_moe = _run_experts(x_moe, x_res, gates.reshape(PSLOTS, 1), W_up, W_down,
                         block_expert.reshape(NPB))
    moe_scat = _sc_scatter_rows(y_moe, toks, S + TRASH)

    aux = (aux1 + aux2).reshape(())
    return moe_scat[:S].reshape(B, S, D), aux


# speedup vs baseline: 1.0241x; 1.0241x over previous
import functools

import jax
import jax.numpy as jnp
import numpy as np
from jax.experimental import pallas as pl
from jax.experimental.pallas import tpu as pltpu
from jax.experimental.pallas import tpu_sc as plsc

B, S, D = 1, 2048, 768
H, DH = 12, 64
DI = 1536
FF = 2048
E = 8
CAP = 1024
EPS = 1e-6
SBLK = 128
NSB = S // SBLK
GCOL = S // SBLK
PBLK = 128
NPB = 24
PSLOTS = NPB * PBLK
TRASH = 128
JT = 256
HPS = 4
F32 = jnp.float32


def _rmsnorm(x, w):
    return x * jax.lax.rsqrt(jnp.mean(x * x, axis=-1, keepdims=True) + EPS) * w


def _cumsum_iorder(x):
    r = jax.lax.broadcasted_iota(jnp.int32, (SBLK, SBLK), 0)
    c = jax.lax.broadcasted_iota(jnp.int32, (SBLK, SBLK), 1)
    tril = (r >= c).astype(F32)
    col = jnp.dot(tril, x, preferred_element_type=F32)
    g = x.shape[1]
    rg = jax.lax.broadcasted_iota(jnp.int32, (g, g), 0)
    cg = jax.lax.broadcasted_iota(jnp.int32, (g, g), 1)
    up = (rg < cg).astype(F32)
    tot = jnp.sum(x, axis=0, keepdims=True)
    pref = jnp.dot(tot, up, preferred_element_type=F32)
    return col + pref


def _ssm_body(h_ref, rms_ref, win_ref, wdt_ref, bdt_ref, alog_ref, wout_ref,
              wmod_ref, h1_ref, wsel_ref, src_ref, aux_ref,
              carry, r_s, winb, wdtb, woutb):
    i = pl.program_id(0)

    @pl.when(i == 0)
    def _():
        carry[...] = jnp.zeros_like(carry)
        winb[...] = win_ref[...].astype(jnp.bfloat16)
        wdtb[...] = wdt_ref[...].astype(jnp.bfloat16)
        woutb[...] = wout_ref[...].astype(jnp.bfloat16)

    @pl.when(i < NSB)
    def _():
        h = h_ref[...]
        xn = _rmsnorm(h, rms_ref[...])
        xnb = xn.astype(jnp.bfloat16)
        xz = jnp.dot(xnb, winb[...], preferred_element_type=F32)
        x_in = xz[:, :DI]
        z = xz[:, DI:]
        delta = jax.nn.softplus(
            jnp.dot(xnb, wdtb[...], preferred_element_type=F32)
            + bdt_ref[...])
        decay = jnp.exp(delta * (-jnp.exp(alog_ref[...])))
        u = delta * x_in

        a, b = decay, u
        d = 1
        while d < SBLK:
            a_s = jnp.concatenate([jnp.ones((d, DI), F32), a[:-d]], axis=0)
            b_s = jnp.concatenate([jnp.zeros((d, DI), F32), b[:-d]], axis=0)
            b = b + a * b_s
            a = a * a_s
            d *= 2
        s = b + a * carry[...]
        carry[...] = s[SBLK - 1:, :]

        h1 = h + jnp.dot((s * jax.nn.silu(z)).astype(jnp.bfloat16),
                         woutb[...], preferred_element_type=F32)
        h1_ref[...] = h1
        rcol = jnp.dot(h1, wmod_ref[...], preferred_element_type=F32)
        for j in range(NSB):
            @pl.when(i == j)
            def _(j=j):
                r_s[:, j:j + 1] = rcol

    @pl.when(i == NSB)
    def _():
        _router_tail(r_s[...], wsel_ref, src_ref, aux_ref)


def _run_ssm(h, rms_ssm, w_in, w_dt, b_dt, a_log, w_out, w_mod):
    full = lambda shp: pl.BlockSpec(shp, lambda i: (0, 0))
    blk = lambda i: (jnp.minimum(i, NSB - 1), 0)
    return pl.pallas_call(
        _ssm_body,
        grid=(NSB + 1,),
        in_specs=[
            pl.BlockSpec((SBLK, D), blk),
            full((1, D)), full((D, 2 * DI)), full((D, DI)), full((1, DI)),
            full((1, DI)), full((DI, D)), full((D, 1)),
        ],
        out_specs=[
            pl.BlockSpec((SBLK, D), blk),
            full((1, CAP)), full((SBLK, GCOL)), full((1, 1)),
        ],
        out_shape=[
            jax.ShapeDtypeStruct((S, D), F32),
            jax.ShapeDtypeStruct((1, CAP), F32),
            jax.ShapeDtypeStruct((SBLK, GCOL), jnp.int32),
            jax.ShapeDtypeStruct((1, 1), F32),
        ],
        scratch_shapes=[pltpu.VMEM((1, DI), F32),
                        pltpu.VMEM((SBLK, NSB), F32),
                        pltpu.VMEM((D, 2 * DI), jnp.bfloat16),
                        pltpu.VMEM((D, DI), jnp.bfloat16),
                        pltpu.VMEM((DI, D), jnp.bfloat16)],
    )(h, rms_ssm.reshape(1, D), w_in, w_dt, b_dt.reshape(1, DI),
      a_log.reshape(1, DI), w_out, w_mod.reshape(D, 1))


def _router_tail(r2, wsel_ref, src_ref, aux_ref):
    u = jax.lax.bitcast_convert_type(r2, jnp.int32)
    keys = u ^ (jax.lax.shift_right_arithmetic(u, 31) & jnp.int32(0x7FFFFFFF))

    def bs_body(_, lh):
        lo, hi = lh
        mid = (lo >> 1) + (hi >> 1) + (lo & hi & 1)
        umid = mid + ((lo ^ hi) & 1)
        cnt = jnp.sum((keys >= umid).astype(jnp.int32))
        ok = cnt >= CAP
        return (jnp.where(ok, umid, lo), jnp.where(ok, hi, umid - 1))

    lo, _ = jax.lax.fori_loop(
        0, 32, bs_body,
        (jnp.int32(np.int32(-2**31)), jnp.int32(np.int32(2**31 - 1))))
    thr = lo

    gt = (keys > thr).astype(F32)
    eq = (keys == thr).astype(F32)
    need = jnp.float32(CAP) - jnp.sum(gt)
    rank_eq = _cumsum_iorder(eq)
    sel = gt + eq * (rank_eq <= need).astype(F32)
    pos = _cumsum_iorder(sel) - 1.0

    icol = (jax.lax.broadcasted_iota(jnp.int32, (SBLK, GCOL), 0)
            + SBLK * jax.lax.broadcasted_iota(jnp.int32, (SBLK, GCOL), 1))
    src_ref[...] = jnp.where(sel > 0.5,
                             jnp.int32(S) + pos.astype(jnp.int32), icol)

    sigv = jax.nn.sigmoid(r2)
    for jt in range(CAP // JT):
        jio = (jax.lax.broadcasted_iota(jnp.int32, (1, JT), 1).astype(F32)
               + jnp.float32(jt * JT))
        acc_w = jnp.zeros((1, JT), F32)
        for g in range(GCOL):
            m = (pos[:, g:g + 1] == jio).astype(F32) * sel[:, g:g + 1]
            acc_w = acc_w + jnp.sum(m * sigv[:, g:g + 1], axis=0,
                                    keepdims=True)
        wsel_ref[0:1, jt * JT:(jt + 1) * JT] = acc_w

    aux_ref[...] = jnp.sum(sigv).reshape(1, 1) / jnp.float32(S)


_NW = 32


def _sc_gather_rows(data, idx_row):
    m = idx_row.shape[1]
    bpw = m // _NW
    mesh = plsc.VectorSubcoreMesh(core_axis_name="c", subcore_axis_name="s")

    @functools.partial(
        pl.kernel,
        out_type=jax.ShapeDtypeStruct((m, D), F32),
        mesh=mesh,
        scratch_types=[
            pltpu.VMEM((bpw,), jnp.int32),
            pltpu.VMEM((bpw, D), F32),
            pltpu.SemaphoreType.DMA,
        ])
    def k(x_hbm, i_hbm, o_hbm, idx_v, rows_v, sem):
        wid = jax.lax.axis_index("s") * 2 + jax.lax.axis_index("c")
        base = wid * bpw
        pltpu.sync_copy(i_hbm.at[pl.ds(base, bpw)], idx_v)
        pltpu.async_copy(x_hbm.at[idx_v], rows_v, sem).wait()
        pltpu.sync_copy(rows_v, o_hbm.at[pl.ds(base, bpw)])

    return k(data, idx_row.reshape(m))


def _sc_gather_rows2(data1, data2, idx_row):
    m = idx_row.shape[1]
    bpw = m // _NW
    mesh = plsc.VectorSubcoreMesh(core_axis_name="c", subcore_axis_name="s")

    @functools.partial(
        pl.kernel,
        out_type=(jax.ShapeDtypeStruct((m, D), F32),
                  jax.ShapeDtypeStruct((m, D), F32)),
        mesh=mesh,
        scratch_types=[
            pltpu.VMEM((bpw,), jnp.int32),
            pltpu.VMEM((bpw, D), F32),
            pltpu.SemaphoreType.DMA,
        ])
    def k(x1_hbm, x2_hbm, i_hbm, o1_hbm, o2_hbm, idx_v, rows_v, sem):
        wid = jax.lax.axis_index("s") * 2 + jax.lax.axis_index("c")
        base = wid * bpw
        pltpu.sync_copy(i_hbm.at[pl.ds(base, bpw)], idx_v)
        pltpu.async_copy(x1_hbm.at[idx_v], rows_v, sem).wait()
        pltpu.sync_copy(rows_v, o1_hbm.at[pl.ds(base, bpw)])
        pltpu.async_copy(x2_hbm.at[idx_v], rows_v, sem).wait()
        pltpu.sync_copy(rows_v, o2_hbm.at[pl.ds(base, bpw)])

    return k(data1, data2, idx_row.reshape(m))


def _sc_scatter_rows(values, idx_row, out_rows):
    m = idx_row.shape[1]
    bpw = m // _NW
    mesh = plsc.VectorSubcoreMesh(core_axis_name="c", subcore_axis_name="s")

    @functools.partial(
        pl.kernel,
        out_type=jax.ShapeDtypeStruct((out_rows, D), F32),
        mesh=mesh,
        scratch_types=[
            pltpu.VMEM((bpw,), jnp.int32),
            pltpu.VMEM((bpw, D), F32),
            pltpu.SemaphoreType.DMA,
        ])
    def k(x_hbm, i_hbm, o_hbm, idx_v, rows_v, sem):
        wid = jax.lax.axis_index("s") * 2 + jax.lax.axis_index("c")
        base = wid * bpw
        pltpu.sync_copy(i_hbm.at[pl.ds(base, bpw)], idx_v)
        pltpu.sync_copy(x_hbm.at[pl.ds(base, bpw)], rows_v)
        pltpu.async_copy(rows_v, o_hbm.at[idx_v], sem).wait()

    return k(values, idx_row.reshape(m))


def _attn_body(h1_ref, src_ref, rms_ref, wq_ref, wk_ref, wv_ref, wo_ref,
               wsel_ref, delta_ref, an_s, acc_s):
    h = pl.program_id(0)

    @pl.when(h == 0)
    def _():
        jrow = jax.lax.broadcasted_iota(jnp.int32, (1, CAP), 1) + jnp.int32(S)
        cols = [
            (src_ref[t] == jrow).astype(jnp.bfloat16) for t in range(GCOL)
        ]
        n = jnp.concatenate(cols, axis=0)
        selb = jax.lax.dot_general(
            n, h1_ref[...].astype(jnp.bfloat16), (((0,), (0,)), ((), ())),
            preferred_element_type=F32)
        an_s[...] = _rmsnorm(selb, rms_ref[...])
        acc_s[...] = jnp.zeros_like(acc_s)

    an = an_s[...].astype(jnp.bfloat16)
    scale = jnp.float32(1.0 / np.sqrt(DH))
    negtri = jnp.where(
        jax.lax.broadcasted_iota(jnp.int32, (SBLK, SBLK), 0)
        >= jax.lax.broadcasted_iota(jnp.int32, (SBLK, SBLK), 1),
        jnp.float32(0.0), jnp.float32(-1e9))
    q2 = jnp.dot(an, wq_ref[...].astype(jnp.bfloat16),
                 preferred_element_type=F32).astype(jnp.bfloat16)
    k2 = jnp.dot(an, wk_ref[...].astype(jnp.bfloat16),
                 preferred_element_type=F32).astype(jnp.bfloat16)
    v2 = jnp.dot(an, wv_ref[...].astype(jnp.bfloat16),
                 preferred_element_type=F32).astype(jnp.bfloat16)
    for j in range(HPS):
        wo_b = wo_ref[j * DH:(j + 1) * DH, :].astype(jnp.bfloat16)
        q = q2[:, j * DH:(j + 1) * DH]
        k = k2[:, j * DH:(j + 1) * DH]
        v = v2[:, j * DH:(j + 1) * DH]
        for qb in range(CAP // SBLK):
            kw = (qb + 1) * SBLK
            qq = q[qb * SBLK:(qb + 1) * SBLK]
            s = jax.lax.dot_general(qq, k[:kw], (((1,), (1,)), ((), ())),
                                    preferred_element_type=F32) * scale
            if qb == 0:
                s = s + negtri
            else:
                s = jnp.concatenate(
                    [s[:, :kw - SBLK], s[:, kw - SBLK:] + negtri], axis=1)
            mx = jnp.max(s, axis=-1, keepdims=True)
            p = jnp.exp(s - mx)
            inv = jnp.float32(1.0) / jnp.sum(p, axis=-1, keepdims=True)
            o = jnp.dot(p.astype(jnp.bfloat16), v[:kw],
                        preferred_element_type=F32) * inv
            acc_s[qb * SBLK:(qb + 1) * SBLK, :] += jnp.dot(
                o.astype(jnp.bfloat16), wo_b, preferred_element_type=F32)

    @pl.when(h == H // HPS - 1)
    def _():
        delta_ref[...] = acc_s[...] * wsel_ref[...]


def _run_attn(h1, src3, rms_attn, wq, wk, wv, wo, wsel_col):
    full = lambda shp: pl.BlockSpec(shp, lambda h: (0, 0))
    return pl.pallas_call(
        _attn_body,
        grid=(H // HPS,),
        in_specs=[
            full((S, D)),
            pl.BlockSpec((GCOL, SBLK, 1), lambda h: (0, 0, 0)),
            full((1, D)),
            pl.BlockSpec((D, HPS * DH), lambda h: (0, h)),
            pl.BlockSpec((D, HPS * DH), lambda h: (0, h)),
            pl.BlockSpec((D, HPS * DH), lambda h: (0, h)),
            pl.BlockSpec((HPS * DH, D), lambda h: (h, 0)),
            full((CAP, 1)),
        ],
        out_specs=full((CAP, D)),
        out_shape=jax.ShapeDtypeStruct((CAP, D), F32),
        scratch_shapes=[pltpu.VMEM((CAP, D), F32), pltpu.VMEM((CAP, D), F32)],
    )(h1, src3, rms_attn.reshape(1, D), wq, wk, wv, wo, wsel_col)


def _gate_body(h1_ref, delta_ref, src_ref, rms_ref, wg_ref, mn_ref, h2_ref,
               tokg_ref, toks_ref, gates_ref, be_ref, aux_ref,
               pe_s, fe_s, e_s, g_s):
    i = pl.program_id(0)

    @pl.when(i == 0)
    def _():
        pe_s[...] = jnp.zeros_like(pe_s)
        fe_s[...] = jnp.zeros_like(fe_s)

    @pl.when(i < NSB)
    def _():
        jrow = jax.lax.broadcasted_iota(jnp.int32, (1, CAP), 1) + jnp.int32(S)
        mcol = (src_ref[0] == jrow).astype(jnp.bfloat16)
        h2 = h1_ref[...] + jnp.dot(mcol, delta_ref[...].astype(jnp.bfloat16),
                                   preferred_element_type=F32)
        h2_ref[...] = h2
        mn = _rmsnorm(h2, rms_ref[...])
        mn_ref[...] = mn
        logits = jnp.dot(mn, wg_ref[...], preferred_element_type=F32)
        mx = jnp.max(logits, axis=-1, keepdims=True)
        ex = jnp.exp(logits - mx)
        probs = ex / jnp.sum(ex, axis=-1, keepdims=True)
        g = jnp.max(probs, axis=-1, keepdims=True)
        ei = jax.lax.broadcasted_iota(jnp.int32, (SBLK, E), 1)
        eid = jnp.min(jnp.where(probs >= g, ei, jnp.int32(E)), axis=-1,
                      keepdims=True)
        for j in range(NSB):
            @pl.when(i == j)
            def _(j=j):
                e_s[:, j:j + 1] = eid
                g_s[:, j:j + 1] = g
        pe_s[...] += jnp.sum(probs, axis=0, keepdims=True)
        fe_s[...] += jnp.sum((ei == eid).astype(F32), axis=0, keepdims=True)

    @pl.when(i == NSB)
    def _():
        aux_ref[...] = (jnp.float32(E) / jnp.float32(S * S)
                        * jnp.sum(fe_s[...] * pe_s[...])).reshape(1, 1)
        _route_tail(e_s[...], g_s[...], tokg_ref, toks_ref, gates_ref, be_ref)


def _run_gate(h1, delta, src3, rms_moe, w_gate):
    full = lambda shp: pl.BlockSpec(shp, lambda i: (0, 0))
    blk = lambda i: (jnp.minimum(i, NSB - 1), 0)
    return pl.pallas_call(
        _gate_body,
        grid=(NSB + 1,),
        in_specs=[pl.BlockSpec((SBLK, D), blk),
                  full((CAP, D)),
                  pl.BlockSpec((1, SBLK, 1), lambda i: (jnp.minimum(i, NSB - 1), 0, 0)),
                  full((1, D)), full((D, E))],
        out_specs=[
            pl.BlockSpec((SBLK, D), blk),
            pl.BlockSpec((SBLK, D), blk),
            full((1, PSLOTS)), full((1, PSLOTS)), full((1, PSLOTS)),
            full((1, NPB)), full((1, 1)),
        ],
        out_shape=[
            jax.ShapeDtypeStruct((S, D), F32),
            jax.ShapeDtypeStruct((S, D), F32),
            jax.ShapeDtypeStruct((1, PSLOTS), jnp.int32),
            jax.ShapeDtypeStruct((1, PSLOTS), jnp.int32),
            jax.ShapeDtypeStruct((1, PSLOTS), F32),
            jax.ShapeDtypeStruct((1, NPB), jnp.int32),
            jax.ShapeDtypeStruct((1, 1), F32),
        ],
        scratch_shapes=[pltpu.VMEM((1, E), F32), pltpu.VMEM((1, E), F32),
                        pltpu.VMEM((SBLK, NSB), jnp.int32),
                        pltpu.VMEM((SBLK, NSB), F32)],
    )(h1, delta, src3, rms_moe.reshape(1, D), w_gate)


def _route_tail(ecol, gcol, tokg_ref, toks_ref, gates_ref, be_ref):
    slot = jnp.zeros((SBLK, GCOL), F32)
    starts, ends, real_ends = [], [], []
    off = jnp.int32(0)
    for e in range(E):
        m = (ecol == e).astype(F32)
        rank = _cumsum_iorder(m)
        cnt = jnp.sum(m).astype(jnp.int32)
        slot = slot + m * (jnp.float32(1.0) * off + rank - 1.0)
        starts.append(off)
        real_ends.append(off + cnt)
        off = off + ((cnt + PBLK - 1) // PBLK) * PBLK
        ends.append(off)

    bio = jax.lax.broadcasted_iota(jnp.int32, (1, NPB), 1)
    be = jnp.full((1, NPB), E - 1, jnp.int32)
    for e in range(E - 1):
        inb = (bio >= starts[e] // PBLK) & (bio < ends[e] // PBLK)
        be = be - jnp.int32(E - 1 - e) * inb.astype(jnp.int32)
    be_ref[...] = be

    for jt in range(PSLOTS // JT):
        jioi = (jax.lax.broadcasted_iota(jnp.int32, (1, JT), 1)
                + jnp.int32(jt * JT))
        jio = jioi.astype(F32)
        acc_t = jnp.zeros((1, JT), F32)
        acc_g = jnp.zeros((1, JT), F32)
        for g in range(GCOL):
            m = (slot[:, g:g + 1] == jio).astype(F32)
            gidx = (jax.lax.broadcasted_iota(jnp.int32, (SBLK, 1), 0)
                    .astype(F32) + jnp.float32(g * SBLK))
            acc_t = acc_t + jnp.sum(m * gidx, axis=0, keepdims=True)
            acc_g = acc_g + jnp.sum(m * gcol[:, g:g + 1], axis=0,
                                    keepdims=True)
        covered = jnp.zeros((1, JT), jnp.bool_)
        for e in range(E):
            covered = covered | ((jioi >= starts[e]) & (jioi < real_ends[e]))
        tok = acc_t.astype(jnp.int32)
        sl = slice(jt * JT, (jt + 1) * JT)
        tokg_ref[0:1, sl] = jnp.where(covered, tok, jioi % S)
        toks_ref[0:1, sl] = jnp.where(covered, tok,
                                      jnp.int32(S) + (jioi % TRASH))
        gates_ref[0:1, sl] = acc_g


def _expert_body(be_ref, x_ref, res_ref, g_ref, wup_ref, wdn_ref, o_ref):
    x = x_ref[...].astype(jnp.bfloat16)
    hmid = jax.nn.silu(jnp.dot(x, wup_ref[0].astype(jnp.bfloat16),
                               preferred_element_type=F32))
    o = jnp.dot(hmid.astype(jnp.bfloat16), wdn_ref[0].astype(jnp.bfloat16),
                preferred_element_type=F32)
    o_ref[...] = res_ref[...] + o * g_ref[...]


def _run_experts(x_moe, x_res, gates_col, w_up, w_down, block_expert):
    spec = pltpu.PrefetchScalarGridSpec(
        num_scalar_prefetch=1,
        grid=(NPB,),
        in_specs=[
            pl.BlockSpec((PBLK, D), lambda i, be: (i, 0)),
            pl.BlockSpec((PBLK, D), lambda i, be: (i, 0)),
            pl.BlockSpec((PBLK, 1), lambda i, be: (i, 0)),
            pl.BlockSpec((1, D, FF), lambda i, be: (be[i], 0, 0)),
            pl.BlockSpec((1, FF, D), lambda i, be: (be[i], 0, 0)),
        ],
        out_specs=pl.BlockSpec((PBLK, D), lambda i, be: (i, 0)),
    )
    return pl.pallas_call(
        _expert_body,
        grid_spec=spec,
        out_shape=jax.ShapeDtypeStruct((PSLOTS, D), F32),
    )(block_expert, x_moe, x_res, gates_col, w_up, w_down)


def kernel(hidden_states, rms_ssm, W_in, W_dt, b_dt, A_log, W_out_ssm, w_mod,
           rms_attn, Wq, Wk, Wv, Wo, rms_moe, W_gate, W_up, W_down):
    h = hidden_states.reshape(S, D)

    h1, wsel_row, src_col, aux1 = _run_ssm(
        h, rms_ssm, W_in, W_dt, b_dt, A_log, W_out_ssm, w_mod)

    src3 = src_col.T.reshape(GCOL, SBLK, 1)
    delta = _run_attn(h1, src3, rms_attn, Wq, Wk, Wv, Wo,
                      wsel_row.reshape(CAP, 1))
    mn, h2, tokg, toks, gates, block_expert, aux2 = _run_gate(
        h1, delta, src3, rms_moe, W_gate)

    x_moe, x_res = _sc_gather_rows2(mn, h2, tokg)
    y_moe = _run_experts(x_moe, x_res, gates.reshape(PSLOTS, 1), W_up, W_down,
                         block_expert.reshape(NPB))
    moe_scat = _sc_scatter_rows(y_moe, toks, S + TRASH)

    aux = (aux1 + aux2).reshape(())
    return moe_scat[:S].reshape(B, S, D), aux


# speedup vs baseline: 1.0290x; 1.0048x over previous
import functools

import jax
import jax.numpy as jnp
import numpy as np
from jax.experimental import pallas as pl
from jax.experimental.pallas import tpu as pltpu
from jax.experimental.pallas import tpu_sc as plsc

B, S, D = 1, 2048, 768
H, DH = 12, 64
DI = 1536
FF = 2048
E = 8
CAP = 1024
EPS = 1e-6
SBLK = 128
NSB = S // SBLK
GCOL = S // SBLK
PBLK = 128
NPB = 24
PSLOTS = NPB * PBLK
TRASH = 128
JT = 256
HPS = 4
F32 = jnp.float32


def _rmsnorm(x, w):
    return x * jax.lax.rsqrt(jnp.mean(x * x, axis=-1, keepdims=True) + EPS) * w


def _cumsum_iorder(x):
    r = jax.lax.broadcasted_iota(jnp.int32, (SBLK, SBLK), 0)
    c = jax.lax.broadcasted_iota(jnp.int32, (SBLK, SBLK), 1)
    tril = (r >= c).astype(F32)
    col = jnp.dot(tril, x, preferred_element_type=F32)
    g = x.shape[1]
    rg = jax.lax.broadcasted_iota(jnp.int32, (g, g), 0)
    cg = jax.lax.broadcasted_iota(jnp.int32, (g, g), 1)
    up = (rg < cg).astype(F32)
    tot = jnp.sum(x, axis=0, keepdims=True)
    pref = jnp.dot(tot, up, preferred_element_type=F32)
    return col + pref


def _ssm_body(h_ref, rms_ref, win_ref, wdt_ref, bdt_ref, alog_ref, wout_ref,
              wmod_ref, h1_ref, wsel_ref, src_ref, aux_ref,
              carry, r_s, winb, wdtb, woutb):
    i = pl.program_id(0)

    @pl.when(i == 0)
    def _():
        carry[...] = jnp.zeros_like(carry)
        winb[...] = win_ref[...].astype(jnp.bfloat16)
        wdtb[...] = wdt_ref[...].astype(jnp.bfloat16)
        woutb[...] = wout_ref[...].astype(jnp.bfloat16)

    @pl.when(i < NSB)
    def _():
        h = h_ref[...]
        xn = _rmsnorm(h, rms_ref[...])
        xnb = xn.astype(jnp.bfloat16)
        xz = jnp.dot(xnb, winb[...], preferred_element_type=F32)
        x_in = xz[:, :DI]
        z = xz[:, DI:]
        delta = jax.nn.softplus(
            jnp.dot(xnb, wdtb[...], preferred_element_type=F32)
            + bdt_ref[...])
        decay = jnp.exp(delta * (-jnp.exp(alog_ref[...])))
        u = delta * x_in

        a, b = decay, u
        d = 1
        while d < SBLK:
            a_s = jnp.concatenate([jnp.ones((d, DI), F32), a[:-d]], axis=0)
            b_s = jnp.concatenate([jnp.zeros((d, DI), F32), b[:-d]], axis=0)
            b = b + a * b_s
            a = a * a_s
            d *= 2
        s = b + a * carry[...]
        carry[...] = s[SBLK - 1:, :]

        h1 = h + jnp.dot((s * jax.nn.silu(z)).astype(jnp.bfloat16),
                         woutb[...], preferred_element_type=F32)
        h1_ref[...] = h1
        rcol = jnp.dot(h1, wmod_ref[...], preferred_element_type=F32)
        for j in range(NSB):
            @pl.when(i == j)
            def _(j=j):
                r_s[:, j:j + 1] = rcol

    @pl.when(i == NSB)
    def _():
        _router_tail(r_s[...], wsel_ref, src_ref, aux_ref)


def _run_ssm(h, rms_ssm, w_in, w_dt, b_dt, a_log, w_out, w_mod):
    full = lambda shp: pl.BlockSpec(shp, lambda i: (0, 0))
    blk = lambda i: (jnp.minimum(i, NSB - 1), 0)
    return pl.pallas_call(
        _ssm_body,
        grid=(NSB + 1,),
        in_specs=[
            pl.BlockSpec((SBLK, D), blk),
            full((1, D)), full((D, 2 * DI)), full((D, DI)), full((1, DI)),
            full((1, DI)), full((DI, D)), full((D, 1)),
        ],
        out_specs=[
            pl.BlockSpec((SBLK, D), blk),
            full((1, CAP)), full((SBLK, GCOL)), full((1, 1)),
        ],
        out_shape=[
            jax.ShapeDtypeStruct((S, D), F32),
            jax.ShapeDtypeStruct((1, CAP), F32),
            jax.ShapeDtypeStruct((SBLK, GCOL), jnp.int32),
            jax.ShapeDtypeStruct((1, 1), F32),
        ],
        scratch_shapes=[pltpu.VMEM((1, DI), F32),
                        pltpu.VMEM((SBLK, NSB), F32),
                        pltpu.VMEM((D, 2 * DI), jnp.bfloat16),
                        pltpu.VMEM((D, DI), jnp.bfloat16),
                        pltpu.VMEM((DI, D), jnp.bfloat16)],
    )(h, rms_ssm.reshape(1, D), w_in, w_dt, b_dt.reshape(1, DI),
      a_log.reshape(1, DI), w_out, w_mod.reshape(D, 1))


def _router_tail(r2, wsel_ref, src_ref, aux_ref):
    u = jax.lax.bitcast_convert_type(r2, jnp.int32)
    keys = u ^ (jax.lax.shift_right_arithmetic(u, 31) & jnp.int32(0x7FFFFFFF))

    def bs_body(_, lh):
        lo, hi = lh
        mid = (lo >> 1) + (hi >> 1) + (lo & hi & 1)
        umid = mid + ((lo ^ hi) & 1)
        cnt = jnp.sum((keys >= umid).astype(jnp.int32))
        ok = cnt >= CAP
        return (jnp.where(ok, umid, lo), jnp.where(ok, hi, umid - 1))

    lo, _ = jax.lax.fori_loop(
        0, 32, bs_body,
        (jnp.int32(np.int32(-2**31)), jnp.int32(np.int32(2**31 - 1))))
    thr = lo

    gt = (keys > thr).astype(F32)
    eq = (keys == thr).astype(F32)
    need = jnp.float32(CAP) - jnp.sum(gt)
    rank_eq = _cumsum_iorder(eq)
    sel = gt + eq * (rank_eq <= need).astype(F32)
    pos = _cumsum_iorder(sel) - 1.0

    icol = (jax.lax.broadcasted_iota(jnp.int32, (SBLK, GCOL), 0)
            + SBLK * jax.lax.broadcasted_iota(jnp.int32, (SBLK, GCOL), 1))
    src_ref[...] = jnp.where(sel > 0.5,
                             jnp.int32(S) + pos.astype(jnp.int32), icol)

    sigv = jax.nn.sigmoid(r2)
    for jt in range(CAP // JT):
        jio = (jax.lax.broadcasted_iota(jnp.int32, (1, JT), 1).astype(F32)
               + jnp.float32(jt * JT))
        mw = jnp.zeros((SBLK, JT), F32)
        for g in range(GCOL):
            m = (pos[:, g:g + 1] == jio).astype(F32) * sel[:, g:g + 1]
            mw = mw + m * sigv[:, g:g + 1]
        wsel_ref[0:1, jt * JT:(jt + 1) * JT] = jnp.sum(mw, axis=0,
                                                      keepdims=True)

    aux_ref[...] = jnp.sum(sigv).reshape(1, 1) / jnp.float32(S)


_NW = 32


def _sc_gather_rows(data, idx_row):
    m = idx_row.shape[1]
    bpw = m // _NW
    mesh = plsc.VectorSubcoreMesh(core_axis_name="c", subcore_axis_name="s")

    @functools.partial(
        pl.kernel,
        out_type=jax.ShapeDtypeStruct((m, D), F32),
        mesh=mesh,
        scratch_types=[
            pltpu.VMEM((bpw,), jnp.int32),
            pltpu.VMEM((bpw, D), F32),
            pltpu.SemaphoreType.DMA,
        ])
    def k(x_hbm, i_hbm, o_hbm, idx_v, rows_v, sem):
        wid = jax.lax.axis_index("s") * 2 + jax.lax.axis_index("c")
        base = wid * bpw
        pltpu.sync_copy(i_hbm.at[pl.ds(base, bpw)], idx_v)
        pltpu.async_copy(x_hbm.at[idx_v], rows_v, sem).wait()
        pltpu.sync_copy(rows_v, o_hbm.at[pl.ds(base, bpw)])

    return k(data, idx_row.reshape(m))


def _sc_gather_rows2(data1, data2, idx_row):
    m = idx_row.shape[1]
    bpw = m // _NW
    mesh = plsc.VectorSubcoreMesh(core_axis_name="c", subcore_axis_name="s")

    @functools.partial(
        pl.kernel,
        out_type=(jax.ShapeDtypeStruct((m, D), F32),
                  jax.ShapeDtypeStruct((m, D), F32)),
        mesh=mesh,
        scratch_types=[
            pltpu.VMEM((bpw,), jnp.int32),
            pltpu.VMEM((bpw, D), F32),
            pltpu.SemaphoreType.DMA,
        ])
    def k(x1_hbm, x2_hbm, i_hbm, o1_hbm, o2_hbm, idx_v, rows_v, sem):
        wid = jax.lax.axis_index("s") * 2 + jax.lax.axis_index("c")
        base = wid * bpw
        pltpu.sync_copy(i_hbm.at[pl.ds(base, bpw)], idx_v)
        pltpu.async_copy(x1_hbm.at[idx_v], rows_v, sem).wait()
        pltpu.sync_copy(rows_v, o1_hbm.at[pl.ds(base, bpw)])
        pltpu.async_copy(x2_hbm.at[idx_v], rows_v, sem).wait()
        pltpu.sync_copy(rows_v, o2_hbm.at[pl.ds(base, bpw)])

    return k(data1, data2, idx_row.reshape(m))


def _sc_scatter_rows(values, idx_row, out_rows):
    m = idx_row.shape[1]
    bpw = m // _NW
    mesh = plsc.VectorSubcoreMesh(core_axis_name="c", subcore_axis_name="s")

    @functools.partial(
        pl.kernel,
        out_type=jax.ShapeDtypeStruct((out_rows, D), F32),
        mesh=mesh,
        scratch_types=[
            pltpu.VMEM((bpw,), jnp.int32),
            pltpu.VMEM((bpw, D), F32),
            pltpu.SemaphoreType.DMA,
        ])
    def k(x_hbm, i_hbm, o_hbm, idx_v, rows_v, sem):
        wid = jax.lax.axis_index("s") * 2 + jax.lax.axis_index("c")
        base = wid * bpw
        pltpu.sync_copy(i_hbm.at[pl.ds(base, bpw)], idx_v)
        pltpu.sync_copy(x_hbm.at[pl.ds(base, bpw)], rows_v)
        pltpu.async_copy(rows_v, o_hbm.at[idx_v], sem).wait()

    return k(values, idx_row.reshape(m))


def _attn_body(h1_ref, src_ref, rms_ref, wq_ref, wk_ref, wv_ref, wo_ref,
               wsel_ref, delta_ref, an_s, acc_s):
    h = pl.program_id(0)

    @pl.when(h == 0)
    def _():
        jrow = jax.lax.broadcasted_iota(jnp.int32, (1, CAP), 1) + jnp.int32(S)
        cols = [
            (src_ref[t] == jrow).astype(jnp.bfloat16) for t in range(GCOL)
        ]
        n = jnp.concatenate(cols, axis=0)
        selb = jax.lax.dot_general(
            n, h1_ref[...].astype(jnp.bfloat16), (((0,), (0,)), ((), ())),
            preferred_element_type=F32)
        an_s[...] = _rmsnorm(selb, rms_ref[...])
        acc_s[...] = jnp.zeros_like(acc_s)

    an = an_s[...].astype(jnp.bfloat16)
    scale = jnp.float32(1.0 / np.sqrt(DH))
    negtri = jnp.where(
        jax.lax.broadcasted_iota(jnp.int32, (SBLK, SBLK), 0)
        >= jax.lax.broadcasted_iota(jnp.int32, (SBLK, SBLK), 1),
        jnp.float32(0.0), jnp.float32(-1e9))
    q2 = jnp.dot(an, wq_ref[...].astype(jnp.bfloat16),
                 preferred_element_type=F32).astype(jnp.bfloat16)
    k2 = jnp.dot(an, wk_ref[...].astype(jnp.bfloat16),
                 preferred_element_type=F32).astype(jnp.bfloat16)
    v2 = jnp.dot(an, wv_ref[...].astype(jnp.bfloat16),
                 preferred_element_type=F32).astype(jnp.bfloat16)
    for j in range(HPS):
        wo_b = wo_ref[j * DH:(j + 1) * DH, :].astype(jnp.bfloat16)
        q = q2[:, j * DH:(j + 1) * DH]
        k = k2[:, j * DH:(j + 1) * DH]
        v = v2[:, j * DH:(j + 1) * DH]
        for qb in range(CAP // SBLK):
            kw = (qb + 1) * SBLK
            qq = q[qb * SBLK:(qb + 1) * SBLK]
            s = jax.lax.dot_general(qq, k[:kw], (((1,), (1,)), ((), ())),
                                    preferred_element_type=F32) * scale
            if qb == 0:
                s = s + negtri
            else:
                s = jnp.concatenate(
                    [s[:, :kw - SBLK], s[:, kw - SBLK:] + negtri], axis=1)
            mx = jnp.max(s, axis=-1, keepdims=True)
            p = jnp.exp(s - mx)
            inv = jnp.float32(1.0) / jnp.sum(p, axis=-1, keepdims=True)
            o = jnp.dot(p.astype(jnp.bfloat16), v[:kw],
                        preferred_element_type=F32) * inv
            acc_s[qb * SBLK:(qb + 1) * SBLK, :] += jnp.dot(
                o.astype(jnp.bfloat16), wo_b, preferred_element_type=F32)

    @pl.when(h == H // HPS - 1)
    def _():
        delta_ref[...] = acc_s[...] * wsel_ref[...]


def _run_attn(h1, src3, rms_attn, wq, wk, wv, wo, wsel_col):
    full = lambda shp: pl.BlockSpec(shp, lambda h: (0, 0))
    return pl.pallas_call(
        _attn_body,
        grid=(H // HPS,),
        in_specs=[
            full((S, D)),
            pl.BlockSpec((GCOL, SBLK, 1), lambda h: (0, 0, 0)),
            full((1, D)),
            pl.BlockSpec((D, HPS * DH), lambda h: (0, h)),
            pl.BlockSpec((D, HPS * DH), lambda h: (0, h)),
            pl.BlockSpec((D, HPS * DH), lambda h: (0, h)),
            pl.BlockSpec((HPS * DH, D), lambda h: (h, 0)),
            full((CAP, 1)),
        ],
        out_specs=full((CAP, D)),
        out_shape=jax.ShapeDtypeStruct((CAP, D), F32),
        scratch_shapes=[pltpu.VMEM((CAP, D), F32), pltpu.VMEM((CAP, D), F32)],
    )(h1, src3, rms_attn.reshape(1, D), wq, wk, wv, wo, wsel_col)


def _gate_body(h1_ref, delta_ref, src_ref, rms_ref, wg_ref, mn_ref, h2_ref,
               tokg_ref, toks_ref, gates_ref, be_ref, aux_ref,
               pe_s, fe_s, e_s, g_s):
    i = pl.program_id(0)

    @pl.when(i == 0)
    def _():
        pe_s[...] = jnp.zeros_like(pe_s)
        fe_s[...] = jnp.zeros_like(fe_s)

    @pl.when(i < NSB)
    def _():
        jrow = jax.lax.broadcasted_iota(jnp.int32, (1, CAP), 1) + jnp.int32(S)
        mcol = (src_ref[0] == jrow).astype(jnp.bfloat16)
        h2 = h1_ref[...] + jnp.dot(mcol, delta_ref[...].astype(jnp.bfloat16),
                                   preferred_element_type=F32)
        h2_ref[...] = h2
        mn = _rmsnorm(h2, rms_ref[...])
        mn_ref[...] = mn
        logits = jnp.dot(mn, wg_ref[...], preferred_element_type=F32)
        mx = jnp.max(logits, axis=-1, keepdims=True)
        ex = jnp.exp(logits - mx)
        probs = ex / jnp.sum(ex, axis=-1, keepdims=True)
        g = jnp.max(probs, axis=-1, keepdims=True)
        ei = jax.lax.broadcasted_iota(jnp.int32, (SBLK, E), 1)
        eid = jnp.min(jnp.where(probs >= g, ei, jnp.int32(E)), axis=-1,
                      keepdims=True)
        for j in range(NSB):
            @pl.when(i == j)
            def _(j=j):
                e_s[:, j:j + 1] = eid
                g_s[:, j:j + 1] = g
        pe_s[...] += jnp.sum(probs, axis=0, keepdims=True)
        fe_s[...] += jnp.sum((ei == eid).astype(F32), axis=0, keepdims=True)

    @pl.when(i == NSB)
    def _():
        aux_ref[...] = (jnp.float32(E) / jnp.float32(S * S)
                        * jnp.sum(fe_s[...] * pe_s[...])).reshape(1, 1)
        _route_tail(e_s[...], g_s[...], tokg_ref, toks_ref, gates_ref, be_ref)


def _run_gate(h1, delta, src3, rms_moe, w_gate):
    full = lambda shp: pl.BlockSpec(shp, lambda i: (0, 0))
    blk = lambda i: (jnp.minimum(i, NSB - 1), 0)
    return pl.pallas_call(
        _gate_body,
        grid=(NSB + 1,),
        in_specs=[pl.BlockSpec((SBLK, D), blk),
                  full((CAP, D)),
                  pl.BlockSpec((1, SBLK, 1), lambda i: (jnp.minimum(i, NSB - 1), 0, 0)),
                  full((1, D)), full((D, E))],
        out_specs=[
            pl.BlockSpec((SBLK, D), blk),
            pl.BlockSpec((SBLK, D), blk),
            full((1, PSLOTS)), full((1, PSLOTS)), full((1, PSLOTS)),
            full((1, NPB)), full((1, 1)),
        ],
        out_shape=[
            jax.ShapeDtypeStruct((S, D), F32),
            jax.ShapeDtypeStruct((S, D), F32),
            jax.ShapeDtypeStruct((1, PSLOTS), jnp.int32),
            jax.ShapeDtypeStruct((1, PSLOTS), jnp.int32),
            jax.ShapeDtypeStruct((1, PSLOTS), F32),
            jax.ShapeDtypeStruct((1, NPB), jnp.int32),
            jax.ShapeDtypeStruct((1, 1), F32),
        ],
        scratch_shapes=[pltpu.VMEM((1, E), F32), pltpu.VMEM((1, E), F32),
                        pltpu.VMEM((SBLK, NSB), jnp.int32),
                        pltpu.VMEM((SBLK, NSB), F32)],
    )(h1, delta, src3, rms_moe.reshape(1, D), w_gate)


def _route_tail(ecol, gcol, tokg_ref, toks_ref, gates_ref, be_ref):
    slot = jnp.zeros((SBLK, GCOL), F32)
    starts, ends, real_ends = [], [], []
    off = jnp.int32(0)
    for e in range(E):
        m = (ecol == e).astype(F32)
        rank = _cumsum_iorder(m)
        cnt = jnp.sum(m).astype(jnp.int32)
        slot = slot + m * (jnp.float32(1.0) * off + rank - 1.0)
        starts.append(off)
        real_ends.append(off + cnt)
        off = off + ((cnt + PBLK - 1) // PBLK) * PBLK
        ends.append(off)

    bio = jax.lax.broadcasted_iota(jnp.int32, (1, NPB), 1)
    be = jnp.full((1, NPB), E - 1, jnp.int32)
    for e in range(E - 1):
        inb = (bio >= starts[e] // PBLK) & (bio < ends[e] // PBLK)
        be = be - jnp.int32(E - 1 - e) * inb.astype(jnp.int32)
    be_ref[...] = be

    for jt in range(PSLOTS // JT):
        jioi = (jax.lax.broadcasted_iota(jnp.int32, (1, JT), 1)
                + jnp.int32(jt * JT))
        jio = jioi.astype(F32)
        mt = jnp.zeros((SBLK, JT), F32)
        mg = jnp.zeros((SBLK, JT), F32)
        for g in range(GCOL):
            m = (slot[:, g:g + 1] == jio).astype(F32)
            gidx = (jax.lax.broadcasted_iota(jnp.int32, (SBLK, 1), 0)
                    .astype(F32) + jnp.float32(g * SBLK))
            mt = mt + m * gidx
            mg = mg + m * gcol[:, g:g + 1]
        acc_t = jnp.sum(mt, axis=0, keepdims=True)
        acc_g = jnp.sum(mg, axis=0, keepdims=True)
        covered = jnp.zeros((1, JT), jnp.bool_)
        for e in range(E):
            covered = covered | ((jioi >= starts[e]) & (jioi < real_ends[e]))
        tok = acc_t.astype(jnp.int32)
        sl = slice(jt * JT, (jt + 1) * JT)
        tokg_ref[0:1, sl] = jnp.where(covered, tok, jioi % S)
        toks_ref[0:1, sl] = jnp.where(covered, tok,
                                      jnp.int32(S) + (jioi % TRASH))
        gates_ref[0:1, sl] = acc_g


def _expert_body(be_ref, x_ref, res_ref, g_ref, wup_ref, wdn_ref, o_ref):
    x = x_ref[...].astype(jnp.bfloat16)
    hmid = jax.nn.silu(jnp.dot(x, wup_ref[0].astype(jnp.bfloat16),
                               preferred_element_type=F32))
    o = jnp.dot(hmid.astype(jnp.bfloat16), wdn_ref[0].astype(jnp.bfloat16),
                preferred_element_type=F32)
    o_ref[...] = res_ref[...] + o * g_ref[...]


def _run_experts(x_moe, x_res, gates_col, w_up, w_down, block_expert):
    spec = pltpu.PrefetchScalarGridSpec(
        num_scalar_prefetch=1,
        grid=(NPB,),
        in_specs=[
            pl.BlockSpec((PBLK, D), lambda i, be: (i, 0)),
            pl.BlockSpec((PBLK, D), lambda i, be: (i, 0)),
            pl.BlockSpec((PBLK, 1), lambda i, be: (i, 0)),
            pl.BlockSpec((1, D, FF), lambda i, be: (be[i], 0, 0)),
            pl.BlockSpec((1, FF, D), lambda i, be: (be[i], 0, 0)),
        ],
        out_specs=pl.BlockSpec((PBLK, D), lambda i, be: (i, 0)),
    )
    return pl.pallas_call(
        _expert_body,
        grid_spec=spec,
        out_shape=jax.ShapeDtypeStruct((PSLOTS, D), F32),
    )(block_expert, x_moe, x_res, gates_col, w_up, w_down)


def kernel(hidden_states, rms_ssm, W_in, W_dt, b_dt, A_log, W_out_ssm, w_mod,
           rms_attn, Wq, Wk, Wv, Wo, rms_moe, W_gate, W_up, W_down):
    h = hidden_states.reshape(S, D)

    h1, wsel_row, src_col, aux1 = _run_ssm(
        h, rms_ssm, W_in, W_dt, b_dt, A_log, W_out_ssm, w_mod)

    src3 = src_col.T.reshape(GCOL, SBLK, 1)
    delta = _run_attn(h1, src3, rms_attn, Wq, Wk, Wv, Wo,
                      wsel_row.reshape(CAP, 1))
    mn, h2, tokg, toks, gates, block_expert, aux2 = _run_gate(
        h1, delta, src3, rms_moe, W_gate)

    x_moe, x_res = _sc_gather_rows2(mn, h2, tokg)
    y_moe = _run_experts(x_moe, x_res, gates.reshape(PSLOTS, 1), W_up, W_down,
                         block_expert.reshape(NPB))
    moe_scat = _sc_scatter_rows(y_moe, toks, S + TRASH)

    aux = (aux1 + aux2).reshape(())
    return moe_scat[:S].reshape(B, S, D), aux


# speedup vs baseline: 1.0470x; 1.0175x over previous
import functools

import jax
import jax.numpy as jnp
import numpy as np
from jax.experimental import pallas as pl
from jax.experimental.pallas import tpu as pltpu
from jax.experimental.pallas import tpu_sc as plsc

B, S, D = 1, 2048, 768
H, DH = 12, 64
DI = 1536
FF = 2048
E = 8
CAP = 1024
EPS = 1e-6
SBLK = 128
NSB = S // SBLK
GCOL = S // SBLK
PBLK = 128
NPB = 24
PSLOTS = NPB * PBLK
TRASH = 128
JT = 256
TBLK = 256
NTB = S // TBLK
HPS = 4
F32 = jnp.float32


def _rmsnorm(x, w):
    return x * jax.lax.rsqrt(jnp.mean(x * x, axis=-1, keepdims=True) + EPS) * w


def _cumsum_iorder(x):
    r = jax.lax.broadcasted_iota(jnp.int32, (SBLK, SBLK), 0)
    c = jax.lax.broadcasted_iota(jnp.int32, (SBLK, SBLK), 1)
    tril = (r >= c).astype(F32)
    col = jnp.dot(tril, x, preferred_element_type=F32)
    g = x.shape[1]
    rg = jax.lax.broadcasted_iota(jnp.int32, (g, g), 0)
    cg = jax.lax.broadcasted_iota(jnp.int32, (g, g), 1)
    up = (rg < cg).astype(F32)
    tot = jnp.sum(x, axis=0, keepdims=True)
    pref = jnp.dot(tot, up, preferred_element_type=F32)
    return col + pref


def _ssm_body(h_ref, rms_ref, win_ref, wdt_ref, bdt_ref, alog_ref, wout_ref,
              wmod_ref, h1_ref, wsel_ref, src_ref, aux_ref,
              carry, r_s, winb, wdtb, woutb):
    i = pl.program_id(0)

    @pl.when(i == 0)
    def _():
        carry[...] = jnp.zeros_like(carry)
        winb[...] = win_ref[...].astype(jnp.bfloat16)
        wdtb[...] = wdt_ref[...].astype(jnp.bfloat16)
        woutb[...] = wout_ref[...].astype(jnp.bfloat16)

    @pl.when(i < NTB)
    def _():
        h = h_ref[...]
        xn = _rmsnorm(h, rms_ref[...])
        xnb = xn.astype(jnp.bfloat16)
        xz = jnp.dot(xnb, winb[...], preferred_element_type=F32)
        x_in = xz[:, :DI]
        z = xz[:, DI:]
        delta = jax.nn.softplus(
            jnp.dot(xnb, wdtb[...], preferred_element_type=F32)
            + bdt_ref[...])
        decay = jnp.exp(delta * (-jnp.exp(alog_ref[...])))
        u = delta * x_in

        def scan_half(a, b, c0):
            d = 1
            while d < SBLK:
                a_s = jnp.concatenate([jnp.ones((d, DI), F32), a[:-d]],
                                      axis=0)
                b_s = jnp.concatenate([jnp.zeros((d, DI), F32), b[:-d]],
                                      axis=0)
                b = b + a * b_s
                a = a * a_s
                d *= 2
            return b + a * c0

        s0 = scan_half(decay[:SBLK], u[:SBLK], carry[...])
        s1 = scan_half(decay[SBLK:], u[SBLK:], s0[SBLK - 1:, :])
        carry[...] = s1[SBLK - 1:, :]
        s = jnp.concatenate([s0, s1], axis=0)

        h1 = h + jnp.dot((s * jax.nn.silu(z)).astype(jnp.bfloat16),
                         woutb[...], preferred_element_type=F32)
        h1_ref[...] = h1
        rcol = jnp.dot(h1, wmod_ref[...], preferred_element_type=F32)
        for j in range(NTB):
            @pl.when(i == j)
            def _(j=j):
                r_s[:, 2 * j:2 * j + 1] = rcol[:SBLK]
                r_s[:, 2 * j + 1:2 * j + 2] = rcol[SBLK:]

    @pl.when(i == NTB)
    def _():
        _router_tail(r_s[...], wsel_ref, src_ref, aux_ref)


def _run_ssm(h, rms_ssm, w_in, w_dt, b_dt, a_log, w_out, w_mod):
    full = lambda shp: pl.BlockSpec(shp, lambda i: (0, 0))
    blk = lambda i: (jnp.minimum(i, NTB - 1), 0)
    return pl.pallas_call(
        _ssm_body,
        grid=(NTB + 1,),
        in_specs=[
            pl.BlockSpec((TBLK, D), blk),
            full((1, D)), full((D, 2 * DI)), full((D, DI)), full((1, DI)),
            full((1, DI)), full((DI, D)), full((D, 1)),
        ],
        out_specs=[
            pl.BlockSpec((TBLK, D), blk),
            full((1, CAP)), full((SBLK, GCOL)), full((1, 1)),
        ],
        out_shape=[
            jax.ShapeDtypeStruct((S, D), F32),
            jax.ShapeDtypeStruct((1, CAP), F32),
            jax.ShapeDtypeStruct((SBLK, GCOL), jnp.int32),
            jax.ShapeDtypeStruct((1, 1), F32),
        ],
        scratch_shapes=[pltpu.VMEM((1, DI), F32),
                        pltpu.VMEM((SBLK, NSB), F32),
                        pltpu.VMEM((D, 2 * DI), jnp.bfloat16),
                        pltpu.VMEM((D, DI), jnp.bfloat16),
                        pltpu.VMEM((DI, D), jnp.bfloat16)],
    )(h, rms_ssm.reshape(1, D), w_in, w_dt, b_dt.reshape(1, DI),
      a_log.reshape(1, DI), w_out, w_mod.reshape(D, 1))


def _router_tail(r2, wsel_ref, src_ref, aux_ref):
    u = jax.lax.bitcast_convert_type(r2, jnp.int32)
    keys = u ^ (jax.lax.shift_right_arithmetic(u, 31) & jnp.int32(0x7FFFFFFF))

    def bs_body(_, lh):
        lo, hi = lh
        mid = (lo >> 1) + (hi >> 1) + (lo & hi & 1)
        umid = mid + ((lo ^ hi) & 1)
        cnt = jnp.sum((keys >= umid).astype(jnp.int32))
        ok = cnt >= CAP
        return (jnp.where(ok, umid, lo), jnp.where(ok, hi, umid - 1))

    lo, _ = jax.lax.fori_loop(
        0, 32, bs_body,
        (jnp.int32(np.int32(-2**31)), jnp.int32(np.int32(2**31 - 1))))
    thr = lo

    gt = (keys > thr).astype(F32)
    eq = (keys == thr).astype(F32)
    need = jnp.float32(CAP) - jnp.sum(gt)
    rank_eq = _cumsum_iorder(eq)
    sel = gt + eq * (rank_eq <= need).astype(F32)
    pos = _cumsum_iorder(sel) - 1.0

    icol = (jax.lax.broadcasted_iota(jnp.int32, (SBLK, GCOL), 0)
            + SBLK * jax.lax.broadcasted_iota(jnp.int32, (SBLK, GCOL), 1))
    src_ref[...] = jnp.where(sel > 0.5,
                             jnp.int32(S) + pos.astype(jnp.int32), icol)

    sigv = jax.nn.sigmoid(r2)
    for jt in range(CAP // JT):
        jio = (jax.lax.broadcasted_iota(jnp.int32, (1, JT), 1).astype(F32)
               + jnp.float32(jt * JT))
        mw = jnp.zeros((SBLK, JT), F32)
        for g in range(GCOL):
            m = (pos[:, g:g + 1] == jio).astype(F32) * sel[:, g:g + 1]
            mw = mw + m * sigv[:, g:g + 1]
        wsel_ref[0:1, jt * JT:(jt + 1) * JT] = jnp.sum(mw, axis=0,
                                                      keepdims=True)

    aux_ref[...] = jnp.sum(sigv).reshape(1, 1) / jnp.float32(S)


_NW = 32


def _sc_gather_rows(data, idx_row):
    m = idx_row.shape[1]
    bpw = m // _NW
    mesh = plsc.VectorSubcoreMesh(core_axis_name="c", subcore_axis_name="s")

    @functools.partial(
        pl.kernel,
        out_type=jax.ShapeDtypeStruct((m, D), F32),
        mesh=mesh,
        scratch_types=[
            pltpu.VMEM((bpw,), jnp.int32),
            pltpu.VMEM((bpw, D), F32),
            pltpu.SemaphoreType.DMA,
        ])
    def k(x_hbm, i_hbm, o_hbm, idx_v, rows_v, sem):
        wid = jax.lax.axis_index("s") * 2 + jax.lax.axis_index("c")
        base = wid * bpw
        pltpu.sync_copy(i_hbm.at[pl.ds(base, bpw)], idx_v)
        pltpu.async_copy(x_hbm.at[idx_v], rows_v, sem).wait()
        pltpu.sync_copy(rows_v, o_hbm.at[pl.ds(base, bpw)])

    return k(data, idx_row.reshape(m))


def _sc_gather_rows2(data1, data2, idx_row):
    m = idx_row.shape[1]
    bpw = m // _NW
    mesh = plsc.VectorSubcoreMesh(core_axis_name="c", subcore_axis_name="s")

    @functools.partial(
        pl.kernel,
        out_type=(jax.ShapeDtypeStruct((m, D), F32),
                  jax.ShapeDtypeStruct((m, D), F32)),
        mesh=mesh,
        scratch_types=[
            pltpu.VMEM((bpw,), jnp.int32),
            pltpu.VMEM((bpw, D), F32),
            pltpu.SemaphoreType.DMA,
        ])
    def k(x1_hbm, x2_hbm, i_hbm, o1_hbm, o2_hbm, idx_v, rows_v, sem):
        wid = jax.lax.axis_index("s") * 2 + jax.lax.axis_index("c")
        base = wid * bpw
        pltpu.sync_copy(i_hbm.at[pl.ds(base, bpw)], idx_v)
        pltpu.async_copy(x1_hbm.at[idx_v], rows_v, sem).wait()
        pltpu.sync_copy(rows_v, o1_hbm.at[pl.ds(base, bpw)])
        pltpu.async_copy(x2_hbm.at[idx_v], rows_v, sem).wait()
        pltpu.sync_copy(rows_v, o2_hbm.at[pl.ds(base, bpw)])

    return k(data1, data2, idx_row.reshape(m))


def _sc_scatter_rows(values, idx_row, out_rows):
    m = idx_row.shape[1]
    bpw = m // _NW
    mesh = plsc.VectorSubcoreMesh(core_axis_name="c", subcore_axis_name="s")

    @functools.partial(
        pl.kernel,
        out_type=jax.ShapeDtypeStruct((out_rows, D), F32),
        mesh=mesh,
        scratch_types=[
            pltpu.VMEM((bpw,), jnp.int32),
            pltpu.VMEM((bpw, D), F32),
            pltpu.SemaphoreType.DMA,
        ])
    def k(x_hbm, i_hbm, o_hbm, idx_v, rows_v, sem):
        wid = jax.lax.axis_index("s") * 2 + jax.lax.axis_index("c")
        base = wid * bpw
        pltpu.sync_copy(i_hbm.at[pl.ds(base, bpw)], idx_v)
        pltpu.sync_copy(x_hbm.at[pl.ds(base, bpw)], rows_v)
        pltpu.async_copy(rows_v, o_hbm.at[idx_v], sem).wait()

    return k(values, idx_row.reshape(m))


def _attn_body(h1_ref, src_ref, rms_ref, wq_ref, wk_ref, wv_ref, wo_ref,
               wsel_ref, delta_ref, an_s, acc_s):
    h = pl.program_id(0)

    @pl.when(h == 0)
    def _():
        jrow = jax.lax.broadcasted_iota(jnp.int32, (1, CAP), 1) + jnp.int32(S)
        cols = [
            (src_ref[t] == jrow).astype(jnp.bfloat16) for t in range(GCOL)
        ]
        n = jnp.concatenate(cols, axis=0)
        selb = jax.lax.dot_general(
            n, h1_ref[...].astype(jnp.bfloat16), (((0,), (0,)), ((), ())),
            preferred_element_type=F32)
        an_s[...] = _rmsnorm(selb, rms_ref[...])
        acc_s[...] = jnp.zeros_like(acc_s)

    an = an_s[...].astype(jnp.bfloat16)
    scale = jnp.float32(1.0 / np.sqrt(DH))
    negtri = jnp.where(
        jax.lax.broadcasted_iota(jnp.int32, (SBLK, SBLK), 0)
        >= jax.lax.broadcasted_iota(jnp.int32, (SBLK, SBLK), 1),
        jnp.float32(0.0), jnp.float32(-1e9))
    q2 = jnp.dot(an, wq_ref[...].astype(jnp.bfloat16),
                 preferred_element_type=F32).astype(jnp.bfloat16)
    k2 = jnp.dot(an, wk_ref[...].astype(jnp.bfloat16),
                 preferred_element_type=F32).astype(jnp.bfloat16)
    v2 = jnp.dot(an, wv_ref[...].astype(jnp.bfloat16),
                 preferred_element_type=F32).astype(jnp.bfloat16)
    for j in range(HPS):
        wo_b = wo_ref[j * DH:(j + 1) * DH, :].astype(jnp.bfloat16)
        q = q2[:, j * DH:(j + 1) * DH]
        k = k2[:, j * DH:(j + 1) * DH]
        v = v2[:, j * DH:(j + 1) * DH]
        for qb in range(CAP // SBLK):
            kw = (qb + 1) * SBLK
            qq = q[qb * SBLK:(qb + 1) * SBLK]
            s = jax.lax.dot_general(qq, k[:kw], (((1,), (1,)), ((), ())),
                                    preferred_element_type=F32) * scale
            if qb == 0:
                s = s + negtri
            else:
                s = jnp.concatenate(
                    [s[:, :kw - SBLK], s[:, kw - SBLK:] + negtri], axis=1)
            mx = jnp.max(s, axis=-1, keepdims=True)
            p = jnp.exp(s - mx)
            inv = jnp.float32(1.0) / jnp.sum(p, axis=-1, keepdims=True)
            o = jnp.dot(p.astype(jnp.bfloat16), v[:kw],
                        preferred_element_type=F32) * inv
            acc_s[qb * SBLK:(qb + 1) * SBLK, :] += jnp.dot(
                o.astype(jnp.bfloat16), wo_b, preferred_element_type=F32)

    @pl.when(h == H // HPS - 1)
    def _():
        delta_ref[...] = acc_s[...] * wsel_ref[...]


def _run_attn(h1, src3, rms_attn, wq, wk, wv, wo, wsel_col):
    full = lambda shp: pl.BlockSpec(shp, lambda h: (0, 0))
    return pl.pallas_call(
        _attn_body,
        grid=(H // HPS,),
        in_specs=[
            full((S, D)),
            pl.BlockSpec((GCOL, SBLK, 1), lambda h: (0, 0, 0)),
            full((1, D)),
            pl.BlockSpec((D, HPS * DH), lambda h: (0, h)),
            pl.BlockSpec((D, HPS * DH), lambda h: (0, h)),
            pl.BlockSpec((D, HPS * DH), lambda h: (0, h)),
            pl.BlockSpec((HPS * DH, D), lambda h: (h, 0)),
            full((CAP, 1)),
        ],
        out_specs=full((CAP, D)),
        out_shape=jax.ShapeDtypeStruct((CAP, D), F32),
        scratch_shapes=[pltpu.VMEM((CAP, D), F32), pltpu.VMEM((CAP, D), F32)],
    )(h1, src3, rms_attn.reshape(1, D), wq, wk, wv, wo, wsel_col)


def _gate_body(h1_ref, delta_ref, src_ref, rms_ref, wg_ref, mn_ref, h2_ref,
               tokg_ref, toks_ref, gates_ref, be_ref, aux_ref,
               pe_s, fe_s, e_s, g_s):
    i = pl.program_id(0)

    @pl.when(i == 0)
    def _():
        pe_s[...] = jnp.zeros_like(pe_s)
        fe_s[...] = jnp.zeros_like(fe_s)

    @pl.when(i < NSB)
    def _():
        jrow = jax.lax.broadcasted_iota(jnp.int32, (1, CAP), 1) + jnp.int32(S)
        mcol = (src_ref[0] == jrow).astype(jnp.bfloat16)
        h2 = h1_ref[...] + jnp.dot(mcol, delta_ref[...].astype(jnp.bfloat16),
                                   preferred_element_type=F32)
        h2_ref[...] = h2
        mn = _rmsnorm(h2, rms_ref[...])
        mn_ref[...] = mn
        logits = jnp.dot(mn, wg_ref[...], preferred_element_type=F32)
        mx = jnp.max(logits, axis=-1, keepdims=True)
        ex = jnp.exp(logits - mx)
        probs = ex / jnp.sum(ex, axis=-1, keepdims=True)
        g = jnp.max(probs, axis=-1, keepdims=True)
        ei = jax.lax.broadcasted_iota(jnp.int32, (SBLK, E), 1)
        eid = jnp.min(jnp.where(probs >= g, ei, jnp.int32(E)), axis=-1,
                      keepdims=True)
        for j in range(NSB):
            @pl.when(i == j)
            def _(j=j):
                e_s[:, j:j + 1] = eid
                g_s[:, j:j + 1] = g
        pe_s[...] += jnp.sum(probs, axis=0, keepdims=True)
        fe_s[...] += jnp.sum((ei == eid).astype(F32), axis=0, keepdims=True)

    @pl.when(i == NSB)
    def _():
        aux_ref[...] = (jnp.float32(E) / jnp.float32(S * S)
                        * jnp.sum(fe_s[...] * pe_s[...])).reshape(1, 1)
        _route_tail(e_s[...], g_s[...], tokg_ref, toks_ref, gates_ref, be_ref)


def _run_gate(h1, delta, src3, rms_moe, w_gate):
    full = lambda shp: pl.BlockSpec(shp, lambda i: (0, 0))
    blk = lambda i: (jnp.minimum(i, NSB - 1), 0)
    return pl.pallas_call(
        _gate_body,
        grid=(NSB + 1,),
        in_specs=[pl.BlockSpec((SBLK, D), blk),
                  full((CAP, D)),
                  pl.BlockSpec((1, SBLK, 1), lambda i: (jnp.minimum(i, NSB - 1), 0, 0)),
                  full((1, D)), full((D, E))],
        out_specs=[
            pl.BlockSpec((SBLK, D), blk),
            pl.BlockSpec((SBLK, D), blk),
            full((1, PSLOTS)), full((1, PSLOTS)), full((1, PSLOTS)),
            full((1, NPB)), full((1, 1)),
        ],
        out_shape=[
            jax.ShapeDtypeStruct((S, D), F32),
            jax.ShapeDtypeStruct((S, D), F32),
            jax.ShapeDtypeStruct((1, PSLOTS), jnp.int32),
            jax.ShapeDtypeStruct((1, PSLOTS), jnp.int32),
            jax.ShapeDtypeStruct((1, PSLOTS), F32),
            jax.ShapeDtypeStruct((1, NPB), jnp.int32),
            jax.ShapeDtypeStruct((1, 1), F32),
        ],
        scratch_shapes=[pltpu.VMEM((1, E), F32), pltpu.VMEM((1, E), F32),
                        pltpu.VMEM((SBLK, NSB), jnp.int32),
                        pltpu.VMEM((SBLK, NSB), F32)],
    )(h1, delta, src3, rms_moe.reshape(1, D), w_gate)


def _route_tail(ecol, gcol, tokg_ref, toks_ref, gates_ref, be_ref):
    slot = jnp.zeros((SBLK, GCOL), F32)
    starts, ends, real_ends = [], [], []
    off = jnp.int32(0)
    for e in range(E):
        m = (ecol == e).astype(F32)
        rank = _cumsum_iorder(m)
        cnt = jnp.sum(m).astype(jnp.int32)
        slot = slot + m * (jnp.float32(1.0) * off + rank - 1.0)
        starts.append(off)
        real_ends.append(off + cnt)
        off = off + ((cnt + PBLK - 1) // PBLK) * PBLK
        ends.append(off)

    bio = jax.lax.broadcasted_iota(jnp.int32, (1, NPB), 1)
    be = jnp.full((1, NPB), E - 1, jnp.int32)
    for e in range(E - 1):
        inb = (bio >= starts[e] // PBLK) & (bio < ends[e] // PBLK)
        be = be - jnp.int32(E - 1 - e) * inb.astype(jnp.int32)
    be_ref[...] = be

    for jt in range(PSLOTS // JT):
        jioi = (jax.lax.broadcasted_iota(jnp.int32, (1, JT), 1)
                + jnp.int32(jt * JT))
        jio = jioi.astype(F32)
        mt = jnp.zeros((SBLK, JT), F32)
        mg = jnp.zeros((SBLK, JT), F32)
        for g in range(GCOL):
            m = (slot[:, g:g + 1] == jio).astype(F32)
            gidx = (jax.lax.broadcasted_iota(jnp.int32, (SBLK, 1), 0)
                    .astype(F32) + jnp.float32(g * SBLK))
            mt = mt + m * gidx
            mg = mg + m * gcol[:, g:g + 1]
        acc_t = jnp.sum(mt, axis=0, keepdims=True)
        acc_g = jnp.sum(mg, axis=0, keepdims=True)
        covered = jnp.zeros((1, JT), jnp.bool_)
        for e in range(E):
            covered = covered | ((jioi >= starts[e]) & (jioi < real_ends[e]))
        tok = acc_t.astype(jnp.int32)
        sl = slice(jt * JT, (jt + 1) * JT)
        tokg_ref[0:1, sl] = jnp.where(covered, tok, jioi % S)
        toks_ref[0:1, sl] = jnp.where(covered, tok,
                                      jnp.int32(S) + (jioi % TRASH))
        gates_ref[0:1, sl] = acc_g


def _expert_body(be_ref, x_ref, res_ref, g_ref, wup_ref, wdn_ref, o_ref):
    x = x_ref[...].astype(jnp.bfloat16)
    hmid = jax.nn.silu(jnp.dot(x, wup_ref[0].astype(jnp.bfloat16),
                               preferred_element_type=F32))
    o = jnp.dot(hmid.astype(jnp.bfloat16), wdn_ref[0].astype(jnp.bfloat16),
                preferred_element_type=F32)
    o_ref[...] = res_ref[...] + o * g_ref[...]


def _run_experts(x_moe, x_res, gates_col, w_up, w_down, block_expert):
    spec = pltpu.PrefetchScalarGridSpec(
        num_scalar_prefetch=1,
        grid=(NPB,),
        in_specs=[
            pl.BlockSpec((PBLK, D), lambda i, be: (i, 0)),
            pl.BlockSpec((PBLK, D), lambda i, be: (i, 0)),
            pl.BlockSpec((PBLK, 1), lambda i, be: (i, 0)),
            pl.BlockSpec((1, D, FF), lambda i, be: (be[i], 0, 0)),
            pl.BlockSpec((1, FF, D), lambda i, be: (be[i], 0, 0)),
        ],
        out_specs=pl.BlockSpec((PBLK, D), lambda i, be: (i, 0)),
    )
    return pl.pallas_call(
        _expert_body,
        grid_spec=spec,
        out_shape=jax.ShapeDtypeStruct((PSLOTS, D), F32),
    )(block_expert, x_moe, x_res, gates_col, w_up, w_down)


def kernel(hidden_states, rms_ssm, W_in, W_dt, b_dt, A_log, W_out_ssm, w_mod,
           rms_attn, Wq, Wk, Wv, Wo, rms_moe, W_gate, W_up, W_down):
    h = hidden_states.reshape(S, D)

    h1, wsel_row, src_col, aux1 = _run_ssm(
        h, rms_ssm, W_in, W_dt, b_dt, A_log, W_out_ssm, w_mod)

    src3 = src_col.T.reshape(GCOL, SBLK, 1)
    delta = _run_attn(h1, src3, rms_attn, Wq, Wk, Wv, Wo,
                      wsel_row.reshape(CAP, 1))
    mn, h2, tokg, toks, gates, block_expert, aux2 = _run_gate(
        h1, delta, src3, rms_moe, W_gate)

    x_moe, x_res = _sc_gather_rows2(mn, h2, tokg)
    y_moe = _run_experts(x_moe, x_res, gates.reshape(PSLOTS, 1), W_up, W_down,
                         block_expert.reshape(NPB))
    moe_scat = _sc_scatter_rows(y_moe, toks, S + TRASH)

    aux = (aux1 + aux2).reshape(())
    return moe_scat[:S].reshape(B, S, D), aux
